# Initial kernel scaffold; baseline (speedup 1.0000x reference)
import jax
import jax.numpy as jnp
from jax.experimental import pallas as pl


def kernel(x, adj, W1, W01, W2, W02):
    raise NotImplementedError("write your pallas kernel here")


# speedup vs baseline: 3.6391x; 3.6391x over previous
import functools

import jax
import jax.numpy as jnp
from jax import lax
from jax.experimental import pallas as pl
from jax.experimental.pallas import tpu as pltpu
from jax.experimental.pallas import tpu_sc as plsc

N = 10000
E = 160000
NFEAT = 256
NHID = 256
NCLASS = 128

_NC = 2
_NS = 16
_C = 128
_H = 40
_EPAD = _NS * 2 * _H * _C
_NPAD = 16 * 632
_RPT = _NPAD // _NS
_F = 128


def _sc_segsum(table_rows, n_stages):
    mesh = plsc.VectorSubcoreMesh(core_axis_name="c", subcore_axis_name="s")

    @functools.partial(
        pl.kernel,
        out_type=jax.ShapeDtypeStruct((_NC, _NPAD, _F), jnp.float32),
        mesh=mesh,
        scratch_types=[
            pltpu.VMEM_SHARED((_NPAD, _F), jnp.float32),
            pltpu.VMEM((_H, 2, _C), jnp.int32),
            pltpu.VMEM((_C, _F), jnp.float32),
            pltpu.VMEM((_C, _F), jnp.float32),
            pltpu.SemaphoreType.DMA,
            pltpu.SemaphoreType.DMA,
        ],
    )
    def seg(table, edges, out, acc, e_v, g0, g1, sem0, sem1):
        c = lax.axis_index("c")
        s = lax.axis_index("s")
        bufs = (g0, g1)
        sems = (sem0, sem1)

        zvec = jnp.zeros((16,), jnp.float32)

        @pl.loop(0, _C)
        def _zrow(r):
            for k in range(_F // 16):
                g0[r, pl.ds(16 * k, 16)] = zvec

        nfull, rem = _RPT // _C, _RPT % _C
        for m in range(nfull):
            pltpu.sync_copy(g0, acc.at[pl.ds(s * _RPT + m * _C, _C)])
        if rem:
            pltpu.sync_copy(g0.at[pl.ds(0, rem)],
                            acc.at[pl.ds(s * _RPT + nfull * _C, rem)])
        plsc.subcore_barrier()

        def start(j, b):
            pltpu.async_copy(table.at[e_v.at[j, 0]], bufs[b], sems[b])

        for stage in range(n_stages):
            pltpu.sync_copy(
                edges.at[c, pl.ds(s * (n_stages * _H) + stage * _H, _H)], e_v)
            start(0, 0)
            start(1, 1)

            @pl.loop(0, _H, step=2)
            def _main(jj):
                for b in range(2):
                    j = jj + b
                    pltpu.make_async_copy(
                        table.at[e_v.at[j, 0]], bufs[b], sems[b]).wait()
                    pltpu.sync_copy(bufs[b], acc.at[e_v.at[j, 1]], add=True)

                    @pl.when(j + 2 < _H)
                    def _():
                        start(j + 2, b)

        plsc.subcore_barrier()
        pltpu.sync_copy(acc.at[pl.ds(s * _RPT, _RPT)],
                        out.at[c, pl.ds(s * _RPT, _RPT)])

    return seg


_BR = 400


def _tc_dense1(agg1, x, W1, W01, W2, W02):
    def body(aL_ref, aR_ref, x_ref, w1_ref, w01_ref, w2_ref, w02_ref,
             g_ref, xw02_ref):
        aL = aL_ref[0]
        aR = aR_ref[0]
        xb = x_ref[...]
        h = aL @ w1_ref[:128, :] + aR @ w1_ref[128:, :] + xb @ w01_ref[...]
        h = jnp.maximum(h, 0.0)
        g_ref[...] = h @ w2_ref[...]
        xw02_ref[...] = xb @ w02_ref[...]

    grid = (N // _BR,)
    return pl.pallas_call(
        body,
        grid=grid,
        in_specs=[
            pl.BlockSpec((1, _BR, 128), lambda i: (0, i, 0)),
            pl.BlockSpec((1, _BR, 128), lambda i: (1, i, 0)),
            pl.BlockSpec((_BR, NFEAT), lambda i: (i, 0)),
            pl.BlockSpec((NFEAT, NHID), lambda i: (0, 0)),
            pl.BlockSpec((NFEAT, NHID), lambda i: (0, 0)),
            pl.BlockSpec((NHID, NCLASS), lambda i: (0, 0)),
            pl.BlockSpec((NFEAT, NCLASS), lambda i: (0, 0)),
        ],
        out_specs=[
            pl.BlockSpec((_BR, NCLASS), lambda i: (i, 0)),
            pl.BlockSpec((_BR, NCLASS), lambda i: (i, 0)),
        ],
        out_shape=[
            jax.ShapeDtypeStruct((N, NCLASS), jnp.float32),
            jax.ShapeDtypeStruct((N, NCLASS), jnp.float32),
        ],
    )(agg1, agg1, x, W1, W01, W2, W02)


def _tc_out(agg2, xw02):
    def body(zL_ref, zR_ref, xw_ref, o_ref):
        z = zL_ref[0] + zR_ref[0] + xw_ref[...]
        m = jnp.max(z, axis=1, keepdims=True)
        e = jnp.exp(z - m)
        lse = jnp.log(jnp.sum(e, axis=1, keepdims=True)) + m
        o_ref[...] = z - lse

    grid = (N // _BR,)
    return pl.pallas_call(
        body,
        grid=grid,
        in_specs=[
            pl.BlockSpec((1, _BR, 128), lambda i: (0, i, 0)),
            pl.BlockSpec((1, _BR, 128), lambda i: (1, i, 0)),
            pl.BlockSpec((_BR, NCLASS), lambda i: (i, 0)),
        ],
        out_specs=pl.BlockSpec((_BR, NCLASS), lambda i: (i, 0)),
        out_shape=jax.ShapeDtypeStruct((N, NCLASS), jnp.float32),
    )(agg2, agg2, xw02)


def kernel(x, adj, W1, W01, W2, W02):
    pad = _EPAD - E
    src_p = jnp.concatenate([adj[0], jnp.zeros((pad,), jnp.int32)])
    dst_p = jnp.concatenate([adj[1], jnp.full((pad,), N, jnp.int32)])
    dst_m = dst_p.reshape(_NS * 2 * _H, _C)

    src2 = src_p * 2
    edges1 = jnp.stack([
        jnp.stack([src2.reshape(_NS * 2 * _H, _C), dst_m], axis=1),
        jnp.stack([(src2 + 1).reshape(_NS * 2 * _H, _C), dst_m], axis=1),
    ])

    edges2 = jnp.stack([src_p.reshape(_NS * 2 * _H, _C), dst_m],
                       axis=1).reshape(2, _NS * _H, 2, _C)

    agg1 = _sc_segsum(2 * N, 2)(x.reshape(2 * N, 128), edges1)

    g, xw02 = _tc_dense1(agg1, x, W1, W01, W2, W02)

    agg2 = _sc_segsum(N, 1)(g, edges2)

    return _tc_out(agg2, xw02)


# speedup vs baseline: 5.8531x; 1.6084x over previous
import functools

import jax
import jax.numpy as jnp
from jax import lax
from jax.experimental import pallas as pl
from jax.experimental.pallas import tpu as pltpu
from jax.experimental.pallas import tpu_sc as plsc

N = 10000
E = 160000
NFEAT = 256
NHID = 256
NCLASS = 128

_NC = 2
_NS = 16
_C = 112
_NCHUNK = 1440
_EPAD = _NCHUNK * _C
_NPAD = 16 * 632
_RPT = _NPAD // _NS
_F = 128


def _sc_segsum(table_rows, n_stages, slen):
    mesh = plsc.VectorSubcoreMesh(core_axis_name="c", subcore_axis_name="s")
    K = n_stages * slen

    @functools.partial(
        pl.kernel,
        out_type=jax.ShapeDtypeStruct((_NC, _NPAD, _F), jnp.float32),
        mesh=mesh,
        scratch_types=[
            pltpu.VMEM_SHARED((_NPAD, _F), jnp.float32),
            pltpu.VMEM((slen, 2, _C), jnp.int32),
            pltpu.VMEM((_C, _F), jnp.float32),
            pltpu.VMEM((_C, _F), jnp.float32),
            pltpu.VMEM((_C, _F), jnp.float32),
            pltpu.SemaphoreType.DMA,
            pltpu.SemaphoreType.DMA,
            pltpu.SemaphoreType.DMA,
            pltpu.SemaphoreType.DMA,
            pltpu.SemaphoreType.DMA,
            pltpu.SemaphoreType.DMA,
        ],
    )
    def seg(table, edges, out, acc, e_v, g0, g1, g2,
            gs0, gs1, gs2, ss0, ss1, ss2):
        c = lax.axis_index("c")
        s = lax.axis_index("s")
        bufs = (g0, g1, g2)
        gsems = (gs0, gs1, gs2)
        ssems = (ss0, ss1, ss2)

        zvec = jnp.zeros((16,), jnp.float32)

        @pl.loop(0, _C)
        def _zrow(r):
            for k in range(_F // 16):
                g0[r, pl.ds(16 * k, 16)] = zvec

        nfull, rem = _RPT // _C, _RPT % _C
        for m in range(nfull):
            pltpu.sync_copy(g0, acc.at[pl.ds(s * _RPT + m * _C, _C)])
        if rem:
            pltpu.sync_copy(g0.at[pl.ds(0, rem)],
                            acc.at[pl.ds(s * _RPT + nfull * _C, rem)])
        plsc.subcore_barrier()

        def start_gather(j, b):
            pltpu.async_copy(table.at[e_v.at[j, 0]], bufs[b], gsems[b])

        def wait_gather(j, b):
            pltpu.make_async_copy(
                table.at[e_v.at[j, 0]], bufs[b], gsems[b]).wait()

        def start_scatter(j, b):
            pltpu.async_copy(bufs[b], acc.at[e_v.at[j, 1]], ssems[b],
                             add=True)

        def wait_scatter(j, b):
            pltpu.make_async_copy(
                bufs[b], acc.at[e_v.at[j, 1]], ssems[b]).wait()

        for stage in range(n_stages):
            pltpu.sync_copy(
                edges.at[c, pl.ds(s * K + stage * slen, slen)], e_v)
            start_gather(0, 0)
            start_gather(1, 1)

            @pl.loop(0, slen, step=3)
            def _main(jj):
                for t in range(3):
                    j = jj + t
                    b = t
                    wait_gather(j, b)
                    start_scatter(j, b)

                    @pl.when(j >= 1)
                    def _():
                        wait_scatter(j - 1, (t + 2) % 3)

                    @pl.when(j + 2 < slen)
                    def _():
                        start_gather(j + 2, (t + 2) % 3)

            wait_scatter(slen - 1, (slen - 1) % 3)

        plsc.subcore_barrier()
        pltpu.sync_copy(acc.at[pl.ds(s * _RPT, _RPT)],
                        out.at[c, pl.ds(s * _RPT, _RPT)])

    return seg


_BR = 400


def _tc_dense1(agg1, x, W1, W01, W2, W02):
    def body(aL_ref, aR_ref, x_ref, w1_ref, w01_ref, w2_ref, w02_ref,
             g_ref, xw02_ref):
        aL = aL_ref[0]
        aR = aR_ref[0]
        xb = x_ref[...]
        h = aL @ w1_ref[:128, :] + aR @ w1_ref[128:, :] + xb @ w01_ref[...]
        h = jnp.maximum(h, 0.0)
        g_ref[...] = h @ w2_ref[...]
        xw02_ref[...] = xb @ w02_ref[...]

    grid = (N // _BR,)
    return pl.pallas_call(
        body,
        grid=grid,
        in_specs=[
            pl.BlockSpec((1, _BR, 128), lambda i: (0, i, 0)),
            pl.BlockSpec((1, _BR, 128), lambda i: (1, i, 0)),
            pl.BlockSpec((_BR, NFEAT), lambda i: (i, 0)),
            pl.BlockSpec((NFEAT, NHID), lambda i: (0, 0)),
            pl.BlockSpec((NFEAT, NHID), lambda i: (0, 0)),
            pl.BlockSpec((NHID, NCLASS), lambda i: (0, 0)),
            pl.BlockSpec((NFEAT, NCLASS), lambda i: (0, 0)),
        ],
        out_specs=[
            pl.BlockSpec((_BR, NCLASS), lambda i: (i, 0)),
            pl.BlockSpec((_BR, NCLASS), lambda i: (i, 0)),
        ],
        out_shape=[
            jax.ShapeDtypeStruct((N, NCLASS), jnp.float32),
            jax.ShapeDtypeStruct((N, NCLASS), jnp.float32),
        ],
    )(agg1, agg1, x, W1, W01, W2, W02)


def _tc_out(agg2, xw02):
    def body(zL_ref, zR_ref, xw_ref, o_ref):
        z = zL_ref[0] + zR_ref[0] + xw_ref[...]
        m = jnp.max(z, axis=1, keepdims=True)
        e = jnp.exp(z - m)
        lse = jnp.log(jnp.sum(e, axis=1, keepdims=True)) + m
        o_ref[...] = z - lse

    grid = (N // _BR,)
    return pl.pallas_call(
        body,
        grid=grid,
        in_specs=[
            pl.BlockSpec((1, _BR, 128), lambda i: (0, i, 0)),
            pl.BlockSpec((1, _BR, 128), lambda i: (1, i, 0)),
            pl.BlockSpec((_BR, NCLASS), lambda i: (i, 0)),
        ],
        out_specs=pl.BlockSpec((_BR, NCLASS), lambda i: (i, 0)),
        out_shape=jax.ShapeDtypeStruct((N, NCLASS), jnp.float32),
    )(agg2, agg2, xw02)


def kernel(x, adj, W1, W01, W2, W02):
    pad = _EPAD - E
    src_p = jnp.concatenate([adj[0], jnp.zeros((pad,), jnp.int32)])
    dst_p = jnp.concatenate([adj[1], jnp.full((pad,), N, jnp.int32)])
    dst_m = dst_p.reshape(_NCHUNK, _C)

    src2 = src_p * 2
    edges1 = jnp.stack([
        jnp.stack([src2.reshape(_NCHUNK, _C), dst_m], axis=1),
        jnp.stack([(src2 + 1).reshape(_NCHUNK, _C), dst_m], axis=1),
    ])

    edges2 = jnp.stack([src_p.reshape(_NCHUNK, _C), dst_m],
                       axis=1).reshape(2, _NCHUNK // 2, 2, _C)

    agg1 = _sc_segsum(2 * N, 5, 18)(x.reshape(2 * N, 128), edges1)

    g, xw02 = _tc_dense1(agg1, x, W1, W01, W2, W02)

    agg2 = _sc_segsum(N, 3, 15)(g, edges2)

    return _tc_out(agg2, xw02)


# speedup vs baseline: 5.9262x; 1.0125x over previous
import functools

import jax
import jax.numpy as jnp
from jax import lax
from jax.experimental import pallas as pl
from jax.experimental.pallas import tpu as pltpu
from jax.experimental.pallas import tpu_sc as plsc

N = 10000
E = 160000
NFEAT = 256
NHID = 256
NCLASS = 128

_NC = 2
_NS = 16
_C = 112
_NCHUNK = 1440
_EPAD = _NCHUNK * _C
_NPAD = 16 * 632
_RPT = _NPAD // _NS
_F = 128


def _sc_segsum(table_rows, n_stages, slen):
    mesh = plsc.VectorSubcoreMesh(core_axis_name="c", subcore_axis_name="s")
    K = n_stages * slen

    @functools.partial(
        pl.kernel,
        out_type=jax.ShapeDtypeStruct((_NC, _NPAD, _F), jnp.float32),
        mesh=mesh,
        scratch_types=[
            pltpu.VMEM_SHARED((_NPAD, _F), jnp.float32),
            pltpu.VMEM((slen, 2, _C), jnp.int32),
            pltpu.VMEM((_C, _F), jnp.float32),
            pltpu.VMEM((_C, _F), jnp.float32),
            pltpu.VMEM((_C, _F), jnp.float32),
            pltpu.SemaphoreType.DMA,
            pltpu.SemaphoreType.DMA,
            pltpu.SemaphoreType.DMA,
            pltpu.SemaphoreType.DMA,
            pltpu.SemaphoreType.DMA,
            pltpu.SemaphoreType.DMA,
        ],
    )
    def seg(table, edges, out, acc, e_v, g0, g1, g2,
            gs0, gs1, gs2, ss0, ss1, ss2):
        c = lax.axis_index("c")
        s = lax.axis_index("s")
        bufs = (g0, g1, g2)
        gsems = (gs0, gs1, gs2)
        ssems = (ss0, ss1, ss2)

        zvec = jnp.zeros((16,), jnp.float32)

        @pl.loop(0, _C)
        def _zrow(r):
            for k in range(_F // 16):
                g0[r, pl.ds(16 * k, 16)] = zvec

        nfull, rem = _RPT // _C, _RPT % _C
        for m in range(nfull):
            pltpu.sync_copy(g0, acc.at[pl.ds(s * _RPT + m * _C, _C)])
        if rem:
            pltpu.sync_copy(g0.at[pl.ds(0, rem)],
                            acc.at[pl.ds(s * _RPT + nfull * _C, rem)])
        plsc.subcore_barrier()

        def start_gather(j, b):
            pltpu.async_copy(table.at[e_v.at[j, 0]], bufs[b], gsems[b])

        def wait_gather(j, b):
            pltpu.make_async_copy(
                table.at[e_v.at[j, 0]], bufs[b], gsems[b]).wait()

        def start_scatter(j, b):
            pltpu.async_copy(bufs[b], acc.at[e_v.at[j, 1]], ssems[b],
                             add=True)

        def wait_scatter(j, b):
            pltpu.make_async_copy(
                bufs[b], acc.at[e_v.at[j, 1]], ssems[b]).wait()

        for stage in range(n_stages):
            pltpu.sync_copy(
                edges.at[c, pl.ds(s * K + stage * slen, slen)], e_v)
            start_gather(0, 0)
            start_gather(1, 1)

            @pl.loop(0, slen, step=3)
            def _main(jj):
                for t in range(3):
                    j = jj + t
                    b = t
                    wait_gather(j, b)
                    start_scatter(j, b)

                    @pl.when(j >= 1)
                    def _():
                        wait_scatter(j - 1, (t + 2) % 3)

                    @pl.when(j + 2 < slen)
                    def _():
                        start_gather(j + 2, (t + 2) % 3)

            wait_scatter(slen - 1, (slen - 1) % 3)

        plsc.subcore_barrier()
        pltpu.sync_copy(acc.at[pl.ds(s * _RPT, _RPT)],
                        out.at[c, pl.ds(s * _RPT, _RPT)])

    return seg


_BR = 400


def _tc_dense1(agg1, x, W1, W01, W2, W02):
    def body(aL_ref, aR_ref, x_ref, w1_ref, w01_ref, w2_ref, w02_ref,
             g_ref, xw02_ref):
        aL = aL_ref[0]
        aR = aR_ref[0]
        xb = x_ref[...]
        h = aL @ w1_ref[:128, :] + aR @ w1_ref[128:, :] + xb @ w01_ref[...]
        h = jnp.maximum(h, 0.0)
        g_ref[...] = h @ w2_ref[...]
        xw02_ref[...] = xb @ w02_ref[...]

    grid = (N // _BR,)
    return pl.pallas_call(
        body,
        grid=grid,
        in_specs=[
            pl.BlockSpec((1, _BR, 128), lambda i: (0, i, 0)),
            pl.BlockSpec((1, _BR, 128), lambda i: (1, i, 0)),
            pl.BlockSpec((_BR, NFEAT), lambda i: (i, 0)),
            pl.BlockSpec((NFEAT, NHID), lambda i: (0, 0)),
            pl.BlockSpec((NFEAT, NHID), lambda i: (0, 0)),
            pl.BlockSpec((NHID, NCLASS), lambda i: (0, 0)),
            pl.BlockSpec((NFEAT, NCLASS), lambda i: (0, 0)),
        ],
        out_specs=[
            pl.BlockSpec((_BR, NCLASS), lambda i: (i, 0)),
            pl.BlockSpec((_BR, NCLASS), lambda i: (i, 0)),
        ],
        out_shape=[
            jax.ShapeDtypeStruct((N, NCLASS), jnp.float32),
            jax.ShapeDtypeStruct((N, NCLASS), jnp.float32),
        ],
    )(agg1, agg1, x, W1, W01, W2, W02)


def _tc_out(agg2, xw02):
    def body(zL_ref, zR_ref, xw_ref, o_ref):
        z = zL_ref[0] + zR_ref[0] + xw_ref[...]
        m = jnp.max(z, axis=1, keepdims=True)
        e = jnp.exp(z - m)
        lse = jnp.log(jnp.sum(e, axis=1, keepdims=True)) + m
        o_ref[...] = z - lse

    grid = (N // _BR,)
    return pl.pallas_call(
        body,
        grid=grid,
        in_specs=[
            pl.BlockSpec((1, _BR, 128), lambda i: (0, i, 0)),
            pl.BlockSpec((1, _BR, 128), lambda i: (1, i, 0)),
            pl.BlockSpec((_BR, NCLASS), lambda i: (i, 0)),
        ],
        out_specs=pl.BlockSpec((_BR, NCLASS), lambda i: (i, 0)),
        out_shape=jax.ShapeDtypeStruct((N, NCLASS), jnp.float32),
    )(agg2, agg2, xw02)


def kernel(x, adj, W1, W01, W2, W02):
    pad = _EPAD - E
    src_p = jnp.concatenate([adj[0], jnp.zeros((pad,), jnp.int32)])
    dummy_dst = N + (jnp.arange(pad, dtype=jnp.int32) % (_NPAD - N))
    dst_p = jnp.concatenate([adj[1], dummy_dst])
    dst_m = dst_p.reshape(_NCHUNK, _C)

    src2 = src_p * 2
    edges1 = jnp.stack([
        jnp.stack([src2.reshape(_NCHUNK, _C), dst_m], axis=1),
        jnp.stack([(src2 + 1).reshape(_NCHUNK, _C), dst_m], axis=1),
    ])

    edges2 = jnp.stack([src_p.reshape(_NCHUNK, _C), dst_m],
                       axis=1).reshape(2, _NCHUNK // 2, 2, _C)

    agg1 = _sc_segsum(2 * N, 5, 18)(x.reshape(2 * N, 128), edges1)

    g, xw02 = _tc_dense1(agg1, x, W1, W01, W2, W02)

    agg2 = _sc_segsum(N, 3, 15)(g, edges2)

    return _tc_out(agg2, xw02)


# speedup vs baseline: 8.6517x; 1.4599x over previous
import functools

import jax
import jax.numpy as jnp
from jax import lax
from jax.experimental import pallas as pl
from jax.experimental.pallas import tpu as pltpu
from jax.experimental.pallas import tpu_sc as plsc

N = 10000
E = 160000
NFEAT = 256
NHID = 256
NCLASS = 128

_NC = 2
_NS = 16
_C = 112
_NCHUNK = 1440
_EPAD = _NCHUNK * _C
_NPAD = 16 * 632
_RPT = _NPAD // _NS
_F = 128


def _sc_segsum(table_rows, n_stages, slen):
    mesh = plsc.VectorSubcoreMesh(core_axis_name="c", subcore_axis_name="s")
    K = n_stages * slen

    @functools.partial(
        pl.kernel,
        out_type=jax.ShapeDtypeStruct((_NC, _NPAD, _F), jnp.float32),
        mesh=mesh,
        scratch_types=[
            pltpu.VMEM_SHARED((_NPAD, _F), jnp.float32),
            pltpu.VMEM((slen, 2, _C), jnp.int32),
            pltpu.VMEM((_C, _F), jnp.float32),
            pltpu.VMEM((_C, _F), jnp.float32),
            pltpu.VMEM((_C, _F), jnp.float32),
            pltpu.SemaphoreType.DMA,
            pltpu.SemaphoreType.DMA,
            pltpu.SemaphoreType.DMA,
            pltpu.SemaphoreType.DMA,
            pltpu.SemaphoreType.DMA,
            pltpu.SemaphoreType.DMA,
        ],
    )
    def seg(table, edges, out, acc, e_v, g0, g1, g2,
            gs0, gs1, gs2, ss0, ss1, ss2):
        c = lax.axis_index("c")
        s = lax.axis_index("s")
        bufs = (g0, g1, g2)
        gsems = (gs0, gs1, gs2)
        ssems = (ss0, ss1, ss2)

        zvec = jnp.zeros((16,), jnp.float32)

        @pl.loop(0, _C)
        def _zrow(r):
            for k in range(_F // 16):
                g0[r, pl.ds(16 * k, 16)] = zvec

        nfull, rem = _RPT // _C, _RPT % _C
        for m in range(nfull):
            pltpu.sync_copy(g0, acc.at[pl.ds(s * _RPT + m * _C, _C)])
        if rem:
            pltpu.sync_copy(g0.at[pl.ds(0, rem)],
                            acc.at[pl.ds(s * _RPT + nfull * _C, rem)])
        plsc.subcore_barrier()

        def start_gather(j, b):
            pltpu.async_copy(table.at[e_v.at[j, 0]], bufs[b], gsems[b])

        def wait_gather(j, b):
            pltpu.make_async_copy(
                table.at[e_v.at[j, 0]], bufs[b], gsems[b]).wait()

        def start_scatter(j, b):
            pltpu.async_copy(bufs[b], acc.at[e_v.at[j, 1]], ssems[b],
                             add=True)

        def wait_scatter(j, b):
            pltpu.make_async_copy(
                bufs[b], acc.at[e_v.at[j, 1]], ssems[b]).wait()

        for stage in range(n_stages):
            pltpu.sync_copy(
                edges.at[c, pl.ds(s * K + stage * slen, slen)], e_v)
            start_gather(0, 0)
            start_gather(1, 1)

            @pl.loop(0, slen, step=3)
            def _main(jj):
                for t in range(3):
                    j = jj + t
                    b = t
                    wait_gather(j, b)
                    start_scatter(j, b)

                    @pl.when(j >= 1)
                    def _():
                        wait_scatter(j - 1, (t + 2) % 3)

                    @pl.when(j + 2 < slen)
                    def _():
                        start_gather(j + 2, (t + 2) % 3)

            wait_scatter(slen - 1, (slen - 1) % 3)

        plsc.subcore_barrier()
        pltpu.sync_copy(acc.at[pl.ds(s * _RPT, _RPT)],
                        out.at[c, pl.ds(s * _RPT, _RPT)])

    return seg


_BR = 400


def _tc_dense1(agg1, x, W1, W01, W2, W02):
    def body(aL_ref, aR_ref, x_ref, w1_ref, w01_ref, w2_ref, w02_ref,
             g_ref, xw02_ref):
        aL = aL_ref[0]
        aR = aR_ref[0]
        xb = x_ref[...]
        h = aL @ w1_ref[:128, :] + aR @ w1_ref[128:, :] + xb @ w01_ref[...]
        h = jnp.maximum(h, 0.0)
        g_ref[...] = h @ w2_ref[...]
        xw02_ref[...] = xb @ w02_ref[...]

    grid = (N // _BR,)
    return pl.pallas_call(
        body,
        grid=grid,
        in_specs=[
            pl.BlockSpec((1, _BR, 128), lambda i: (0, i, 0)),
            pl.BlockSpec((1, _BR, 128), lambda i: (1, i, 0)),
            pl.BlockSpec((_BR, NFEAT), lambda i: (i, 0)),
            pl.BlockSpec((NFEAT, NHID), lambda i: (0, 0)),
            pl.BlockSpec((NFEAT, NHID), lambda i: (0, 0)),
            pl.BlockSpec((NHID, NCLASS), lambda i: (0, 0)),
            pl.BlockSpec((NFEAT, NCLASS), lambda i: (0, 0)),
        ],
        out_specs=[
            pl.BlockSpec((_BR, NCLASS), lambda i: (i, 0)),
            pl.BlockSpec((_BR, NCLASS), lambda i: (i, 0)),
        ],
        out_shape=[
            jax.ShapeDtypeStruct((N, NCLASS), jnp.float32),
            jax.ShapeDtypeStruct((N, NCLASS), jnp.float32),
        ],
    )(agg1, agg1, x, W1, W01, W2, W02)


def _tc_out(agg2, xw02):
    def body(zL_ref, zR_ref, xw_ref, o_ref):
        z = zL_ref[0] + zR_ref[0] + xw_ref[...]
        m = jnp.max(z, axis=1, keepdims=True)
        e = jnp.exp(z - m)
        lse = jnp.log(jnp.sum(e, axis=1, keepdims=True)) + m
        o_ref[...] = z - lse

    grid = (N // _BR,)
    return pl.pallas_call(
        body,
        grid=grid,
        in_specs=[
            pl.BlockSpec((1, _BR, 128), lambda i: (0, i, 0)),
            pl.BlockSpec((1, _BR, 128), lambda i: (1, i, 0)),
            pl.BlockSpec((_BR, NCLASS), lambda i: (i, 0)),
        ],
        out_specs=pl.BlockSpec((_BR, NCLASS), lambda i: (i, 0)),
        out_shape=jax.ShapeDtypeStruct((N, NCLASS), jnp.float32),
    )(agg2, agg2, xw02)


def kernel(x, adj, W1, W01, W2, W02):
    pad = _EPAD - E
    ar = jnp.arange(pad, dtype=jnp.int32)
    src_p = jnp.concatenate([adj[0], ar * 7 % N])
    dst_p = jnp.concatenate([adj[1], N + ar % (_NPAD - N)])
    dst_m = dst_p.reshape(_NCHUNK, _C)

    src2 = src_p * 2
    edges1 = jnp.stack([
        jnp.stack([src2.reshape(_NCHUNK, _C), dst_m], axis=1),
        jnp.stack([(src2 + 1).reshape(_NCHUNK, _C), dst_m], axis=1),
    ])

    edges2 = jnp.stack([src_p.reshape(_NCHUNK, _C), dst_m],
                       axis=1).reshape(2, _NCHUNK // 2, 2, _C)

    agg1 = _sc_segsum(2 * N, 5, 18)(x.reshape(2 * N, 128), edges1)

    g, xw02 = _tc_dense1(agg1, x, W1, W01, W2, W02)

    agg2 = _sc_segsum(N, 3, 15)(g, edges2)

    return _tc_out(agg2, xw02)


# speedup vs baseline: 8.8750x; 1.0258x over previous
import functools

import jax
import jax.numpy as jnp
from jax import lax
from jax.experimental import pallas as pl
from jax.experimental.pallas import tpu as pltpu
from jax.experimental.pallas import tpu_sc as plsc

N = 10000
E = 160000
NFEAT = 256
NHID = 256
NCLASS = 128

_NC = 2
_NS = 16
_C = 112
_NCHUNK = 1440
_EPAD = _NCHUNK * _C
_NPAD = 16 * 632
_RPT = _NPAD // _NS
_F = 128


def _sc_segsum(n_stages, slen, per_core_edges, interleave):
    mesh = plsc.VectorSubcoreMesh(core_axis_name="c", subcore_axis_name="s")
    K = n_stages * slen

    @functools.partial(
        pl.kernel,
        out_type=jax.ShapeDtypeStruct((_NC, _NPAD, _F), jnp.float32),
        mesh=mesh,
        scratch_types=[
            pltpu.VMEM_SHARED((_NPAD, _F), jnp.float32),
            pltpu.VMEM((slen, 2, _C), jnp.int32),
            pltpu.VMEM((_C, _F), jnp.float32),
            pltpu.VMEM((_C, _F), jnp.float32),
            pltpu.VMEM((_C, _F), jnp.float32),
            pltpu.SemaphoreType.DMA,
            pltpu.SemaphoreType.DMA,
            pltpu.SemaphoreType.DMA,
            pltpu.SemaphoreType.DMA,
            pltpu.SemaphoreType.DMA,
            pltpu.SemaphoreType.DMA,
        ],
    )
    def seg(table, edges, out, acc, e_v, g0, g1, g2,
            gs0, gs1, gs2, ss0, ss1, ss2):
        c = lax.axis_index("c")
        s = lax.axis_index("s")
        bufs = (g0, g1, g2)
        gsems = (gs0, gs1, gs2)
        ssems = (ss0, ss1, ss2)

        zvec = jnp.zeros((16,), jnp.float32)

        @pl.loop(0, _C)
        def _zrow(r):
            for k in range(_F // 16):
                g0[r, pl.ds(16 * k, 16)] = zvec

        nfull, rem = _RPT // _C, _RPT % _C
        for m in range(nfull):
            pltpu.sync_copy(g0, acc.at[pl.ds(s * _RPT + m * _C, _C)])
        if rem:
            pltpu.sync_copy(g0.at[pl.ds(0, rem)],
                            acc.at[pl.ds(s * _RPT + nfull * _C, rem)])
        plsc.subcore_barrier()

        def start_gather(j, b):
            pltpu.async_copy(table.at[e_v.at[j, 0]], bufs[b], gsems[b])

        def wait_gather(j, b):
            pltpu.make_async_copy(
                table.at[e_v.at[j, 0]], bufs[b], gsems[b]).wait()

        def start_scatter(j, b):
            pltpu.async_copy(bufs[b], acc.at[e_v.at[j, 1]], ssems[b],
                             add=True)

        def wait_scatter(j, b):
            pltpu.make_async_copy(
                bufs[b], acc.at[e_v.at[j, 1]], ssems[b]).wait()

        cvec = jnp.full((16,), c, dtype=jnp.int32)

        for stage in range(n_stages):
            sl_chunks = pl.ds(s * K + stage * slen, slen)
            if per_core_edges:
                pltpu.sync_copy(edges.at[c, sl_chunks], e_v)
            else:
                pltpu.sync_copy(edges.at[sl_chunks], e_v)

            if interleave:
                @pl.loop(0, slen)
                def _tx(j):
                    for k in range(_C // 16):
                        sl = pl.ds(16 * k, 16)
                        e_v[j, 0, sl] = e_v[j, 0, sl] * 2 + cvec

            start_gather(0, 0)
            start_gather(1, 1)

            @pl.loop(0, slen, step=3)
            def _main(jj):
                for t in range(3):
                    j = jj + t
                    wait_gather(j, t)
                    start_scatter(j, t)

                    @pl.when(j >= 1)
                    def _():
                        wait_scatter(j - 1, (t + 2) % 3)

                    @pl.when(j + 2 < slen)
                    def _():
                        start_gather(j + 2, (t + 2) % 3)

            wait_scatter(slen - 1, (slen - 1) % 3)

        plsc.subcore_barrier()
        pltpu.sync_copy(acc.at[pl.ds(s * _RPT, _RPT)],
                        out.at[c, pl.ds(s * _RPT, _RPT)])

    return seg


_BR = 400


def _tc_pre(x, W01, W02):
    def body(x_ref, w01_ref, w02_ref, p01_ref, xw02_ref):
        xb = x_ref[...]
        p01_ref[...] = xb @ w01_ref[...]
        xw02_ref[...] = xb @ w02_ref[...]

    return pl.pallas_call(
        body,
        grid=(N // _BR,),
        in_specs=[
            pl.BlockSpec((_BR, NFEAT), lambda i: (i, 0)),
            pl.BlockSpec((NFEAT, NHID), lambda i: (0, 0)),
            pl.BlockSpec((NFEAT, NCLASS), lambda i: (0, 0)),
        ],
        out_specs=[
            pl.BlockSpec((_BR, NHID), lambda i: (i, 0)),
            pl.BlockSpec((_BR, NCLASS), lambda i: (i, 0)),
        ],
        out_shape=[
            jax.ShapeDtypeStruct((N, NHID), jnp.float32),
            jax.ShapeDtypeStruct((N, NCLASS), jnp.float32),
        ],
    )(x, W01, W02)


def _tc_mid(agg1, p01, W1, W2):
    def body(aL_ref, aR_ref, p01_ref, w1_ref, w2_ref, g_ref):
        h = (aL_ref[0] @ w1_ref[:128, :] + aR_ref[0] @ w1_ref[128:, :]
             + p01_ref[...])
        h = jnp.maximum(h, 0.0)
        g_ref[...] = h @ w2_ref[...]

    return pl.pallas_call(
        body,
        grid=(N // _BR,),
        in_specs=[
            pl.BlockSpec((1, _BR, 128), lambda i: (0, i, 0)),
            pl.BlockSpec((1, _BR, 128), lambda i: (1, i, 0)),
            pl.BlockSpec((_BR, NHID), lambda i: (i, 0)),
            pl.BlockSpec((NFEAT, NHID), lambda i: (0, 0)),
            pl.BlockSpec((NHID, NCLASS), lambda i: (0, 0)),
        ],
        out_specs=pl.BlockSpec((_BR, NCLASS), lambda i: (i, 0)),
        out_shape=jax.ShapeDtypeStruct((N, NCLASS), jnp.float32),
    )(agg1, agg1, p01, W1, W2)


def _tc_out(agg2, xw02):
    def body(zL_ref, zR_ref, xw_ref, o_ref):
        z = zL_ref[0] + zR_ref[0] + xw_ref[...]
        m = jnp.max(z, axis=1, keepdims=True)
        e = jnp.exp(z - m)
        lse = jnp.log(jnp.sum(e, axis=1, keepdims=True)) + m
        o_ref[...] = z - lse

    return pl.pallas_call(
        body,
        grid=(N // _BR,),
        in_specs=[
            pl.BlockSpec((1, _BR, 128), lambda i: (0, i, 0)),
            pl.BlockSpec((1, _BR, 128), lambda i: (1, i, 0)),
            pl.BlockSpec((_BR, NCLASS), lambda i: (i, 0)),
        ],
        out_specs=pl.BlockSpec((_BR, NCLASS), lambda i: (i, 0)),
        out_shape=jax.ShapeDtypeStruct((N, NCLASS), jnp.float32),
    )(agg2, agg2, xw02)


def kernel(x, adj, W1, W01, W2, W02):
    pad = _EPAD - E
    ar = jnp.arange(pad, dtype=jnp.int32)
    src_p = jnp.concatenate([adj[0], ar * 7 % N])
    dst_p = jnp.concatenate([adj[1], N + ar % (_NPAD - N)])
    edges = jnp.stack([src_p.reshape(_NCHUNK, _C),
                       dst_p.reshape(_NCHUNK, _C)], axis=1)

    p01, xw02 = _tc_pre(x, W01, W02)

    agg1 = _sc_segsum(5, 18, False, True)(x.reshape(2 * N, 128), edges)

    g = _tc_mid(agg1, p01, W1, W2)

    agg2 = _sc_segsum(3, 15, True, False)(
        g, edges.reshape(2, _NCHUNK // 2, 2, _C))

    return _tc_out(agg2, xw02)


# speedup vs baseline: 8.8789x; 1.0004x over previous
import functools

import jax
import jax.numpy as jnp
from jax import lax
from jax.experimental import pallas as pl
from jax.experimental.pallas import tpu as pltpu
from jax.experimental.pallas import tpu_sc as plsc

N = 10000
E = 160000
NFEAT = 256
NHID = 256
NCLASS = 128

_NC = 2
_NS = 16
_C = 112
_NCHUNK = 1440
_EPAD = _NCHUNK * _C
_NPAD = 16 * 632
_RPT = _NPAD // _NS
_F = 128


def _sc_segsum(n_stages, slen, per_core_edges, interleave):
    mesh = plsc.VectorSubcoreMesh(core_axis_name="c", subcore_axis_name="s")
    K = n_stages * slen

    @functools.partial(
        pl.kernel,
        out_type=jax.ShapeDtypeStruct((_NC, _NPAD, _F), jnp.float32),
        mesh=mesh,
        scratch_types=[
            pltpu.VMEM_SHARED((_NPAD, _F), jnp.float32),
            pltpu.VMEM((slen, 2, _C), jnp.int32),
            pltpu.VMEM((_C, _F), jnp.float32),
            pltpu.VMEM((_C, _F), jnp.float32),
            pltpu.VMEM((_C, _F), jnp.float32),
            pltpu.SemaphoreType.DMA,
            pltpu.SemaphoreType.DMA,
            pltpu.SemaphoreType.DMA,
            pltpu.SemaphoreType.DMA,
            pltpu.SemaphoreType.DMA,
            pltpu.SemaphoreType.DMA,
        ],
    )
    def seg(table, edges, out, acc, e_v, g0, g1, g2,
            gs0, gs1, gs2, ss0, ss1, ss2):
        c = lax.axis_index("c")
        s = lax.axis_index("s")
        bufs = (g0, g1, g2)
        gsems = (gs0, gs1, gs2)
        ssems = (ss0, ss1, ss2)

        zvec = jnp.zeros((16,), jnp.float32)

        @pl.loop(0, _C)
        def _zrow(r):
            for k in range(_F // 16):
                g0[r, pl.ds(16 * k, 16)] = zvec

        nfull, rem = _RPT // _C, _RPT % _C
        for m in range(nfull):
            pltpu.sync_copy(g0, acc.at[pl.ds(s * _RPT + m * _C, _C)])
        if rem:
            pltpu.sync_copy(g0.at[pl.ds(0, rem)],
                            acc.at[pl.ds(s * _RPT + nfull * _C, rem)])
        plsc.subcore_barrier()

        def start_gather(j, b):
            pltpu.async_copy(table.at[e_v.at[j, 0]], bufs[b], gsems[b])

        def wait_gather(j, b):
            pltpu.make_async_copy(
                table.at[e_v.at[j, 0]], bufs[b], gsems[b]).wait()

        def start_scatter(j, b):
            pltpu.async_copy(bufs[b], acc.at[e_v.at[j, 1]], ssems[b],
                             add=True)

        def wait_scatter(j, b):
            pltpu.make_async_copy(
                bufs[b], acc.at[e_v.at[j, 1]], ssems[b]).wait()

        cvec = jnp.full((16,), c, dtype=jnp.int32)

        for stage in range(n_stages):
            sl_chunks = pl.ds(s * K + stage * slen, slen)
            if per_core_edges:
                pltpu.sync_copy(edges.at[c, sl_chunks], e_v)
            else:
                pltpu.sync_copy(edges.at[sl_chunks], e_v)

            if interleave:
                @pl.loop(0, slen)
                def _tx(j):
                    for k in range(_C // 16):
                        sl = pl.ds(16 * k, 16)
                        e_v[j, 0, sl] = e_v[j, 0, sl] * 2 + cvec

            start_gather(0, 0)
            start_gather(1, 1)

            @pl.loop(0, slen, step=3)
            def _main(jj):
                for t in range(3):
                    j = jj + t
                    wait_gather(j, t)
                    start_scatter(j, t)

                    @pl.when(j >= 1)
                    def _():
                        wait_scatter(j - 1, (t + 2) % 3)

                    @pl.when(j + 2 < slen)
                    def _():
                        start_gather(j + 2, (t + 2) % 3)

            wait_scatter(slen - 1, (slen - 1) % 3)

        plsc.subcore_barrier()
        pltpu.sync_copy(acc.at[pl.ds(s * _RPT, _RPT)],
                        out.at[c, pl.ds(s * _RPT, _RPT)])

    return seg


_BR = 400


def _tc_pre(x, W01, W02):
    def body(x_ref, w01_ref, w02_ref, p01_ref, xw02_ref):
        xb = x_ref[...]
        p01_ref[...] = xb @ w01_ref[...]
        xw02_ref[...] = xb @ w02_ref[...]

    return pl.pallas_call(
        body,
        grid=(N // _BR,),
        in_specs=[
            pl.BlockSpec((_BR, NFEAT), lambda i: (i, 0)),
            pl.BlockSpec((NFEAT, NHID), lambda i: (0, 0)),
            pl.BlockSpec((NFEAT, NCLASS), lambda i: (0, 0)),
        ],
        out_specs=[
            pl.BlockSpec((_BR, NHID), lambda i: (i, 0)),
            pl.BlockSpec((_BR, NCLASS), lambda i: (i, 0)),
        ],
        out_shape=[
            jax.ShapeDtypeStruct((N, NHID), jnp.float32),
            jax.ShapeDtypeStruct((N, NCLASS), jnp.float32),
        ],
    )(x, W01, W02)


def _tc_mid(agg1, p01, W1, W2):
    def body(aL_ref, aR_ref, p01_ref, w1_ref, w2_ref, g_ref):
        bf = jnp.bfloat16
        f32 = jnp.float32
        h = (jnp.dot(aL_ref[0].astype(bf), w1_ref[:128, :].astype(bf),
                     preferred_element_type=f32)
             + jnp.dot(aR_ref[0].astype(bf), w1_ref[128:, :].astype(bf),
                       preferred_element_type=f32)
             + p01_ref[...])
        h = jnp.maximum(h, 0.0)
        g_ref[...] = jnp.dot(h.astype(bf), w2_ref[...].astype(bf),
                             preferred_element_type=f32)

    return pl.pallas_call(
        body,
        grid=(N // _BR,),
        in_specs=[
            pl.BlockSpec((1, _BR, 128), lambda i: (0, i, 0)),
            pl.BlockSpec((1, _BR, 128), lambda i: (1, i, 0)),
            pl.BlockSpec((_BR, NHID), lambda i: (i, 0)),
            pl.BlockSpec((NFEAT, NHID), lambda i: (0, 0)),
            pl.BlockSpec((NHID, NCLASS), lambda i: (0, 0)),
        ],
        out_specs=pl.BlockSpec((_BR, NCLASS), lambda i: (i, 0)),
        out_shape=jax.ShapeDtypeStruct((N, NCLASS), jnp.float32),
    )(agg1, agg1, p01, W1, W2)


def _tc_out(agg2, xw02):
    def body(zL_ref, zR_ref, xw_ref, o_ref):
        z = zL_ref[0] + zR_ref[0] + xw_ref[...]
        m = jnp.max(z, axis=1, keepdims=True)
        e = jnp.exp(z - m)
        lse = jnp.log(jnp.sum(e, axis=1, keepdims=True)) + m
        o_ref[...] = z - lse

    return pl.pallas_call(
        body,
        grid=(N // _BR,),
        in_specs=[
            pl.BlockSpec((1, _BR, 128), lambda i: (0, i, 0)),
            pl.BlockSpec((1, _BR, 128), lambda i: (1, i, 0)),
            pl.BlockSpec((_BR, NCLASS), lambda i: (i, 0)),
        ],
        out_specs=pl.BlockSpec((_BR, NCLASS), lambda i: (i, 0)),
        out_shape=jax.ShapeDtypeStruct((N, NCLASS), jnp.float32),
    )(agg2, agg2, xw02)


def kernel(x, adj, W1, W01, W2, W02):
    pad = _EPAD - E
    ar = jnp.arange(pad, dtype=jnp.int32)
    src_p = jnp.concatenate([adj[0], ar * 7 % N])
    dst_p = jnp.concatenate([adj[1], N + ar % (_NPAD - N)])
    edges = jnp.stack([src_p.reshape(_NCHUNK, _C),
                       dst_p.reshape(_NCHUNK, _C)], axis=1)

    p01, xw02 = _tc_pre(x, W01, W02)

    agg1 = _sc_segsum(5, 18, False, True)(x.reshape(2 * N, 128), edges)

    g = _tc_mid(agg1, p01, W1, W2)

    agg2 = _sc_segsum(3, 15, True, False)(
        g, edges.reshape(2, _NCHUNK // 2, 2, _C))

    return _tc_out(agg2, xw02)
